# Initial kernel scaffold; baseline (speedup 1.0000x reference)
import jax
import jax.numpy as jnp
from jax.experimental import pallas as pl


def kernel(features, edge_index, W0, b0, W1, b1, W2, b2):
    raise NotImplementedError("write your pallas kernel here")


# speedup vs baseline: 6.8899x; 6.8899x over previous
import functools

import jax
import jax.numpy as jnp
from jax import lax
from jax.experimental import pallas as pl
from jax.experimental.pallas import tpu as pltpu
from jax.experimental.pallas import tpu_sc as plsc

N_NODES = 10000
N_EDGES = 320000
D = 128

NC = 2
NS = 16
NW = NC * NS

EDGES_PER_W = N_EDGES // NW
K = 80
C = EDGES_PER_W // K

RPT = 632
NP = NS * RPT


def _agg_body(compute_deg, x_hbm, src_hbm, dst_hbm, m_out, deg_out,
              acc_sh, deg_sh, src_v, dst_v, rows_v, ones_v, sem):
    cid = lax.axis_index("c")
    sid = lax.axis_index("s")
    wid = sid * NC + cid

    @pl.loop(0, K)
    def _(i):
        for j in range(D // 16):
            rows_v[i, pl.ds(j * 16, 16)] = jnp.zeros((16,), jnp.float32)

    for t in range(7):
        pltpu.sync_copy(rows_v, acc_sh.at[pl.ds(sid * RPT + t * K, K)])
    pltpu.sync_copy(rows_v.at[pl.ds(0, 72)],
                    acc_sh.at[pl.ds(sid * RPT + 560, 72)])
    if compute_deg:
        for t in range(4):
            pltpu.sync_copy(rows_v.at[0], deg_sh.at[pl.ds(sid * RPT + t * 128, 128)])
        pltpu.sync_copy(rows_v.at[0, pl.ds(0, 120)],
                        deg_sh.at[pl.ds(sid * RPT + 512, 120)])
        for j in range(K // 16):
            ones_v[pl.ds(j * 16, 16)] = jnp.ones((16,), jnp.float32)

    pltpu.sync_copy(src_hbm.at[wid], src_v)
    pltpu.sync_copy(dst_hbm.at[wid], dst_v)

    plsc.subcore_barrier()

    @pl.loop(0, C)
    def _(j):
        pltpu.async_copy(x_hbm.at[src_v.at[j]], rows_v, sem).wait()
        pltpu.sync_copy(rows_v, acc_sh.at[dst_v.at[j]], add=True)
        if compute_deg:
            pltpu.sync_copy(ones_v, deg_sh.at[dst_v.at[j]], add=True)

    plsc.subcore_barrier()

    pltpu.sync_copy(acc_sh.at[pl.ds(sid * RPT, RPT)],
                    m_out.at[cid, pl.ds(sid * RPT, RPT)])
    if compute_deg:
        @pl.when(sid == 0)
        def _():
            pltpu.sync_copy(deg_sh, deg_out.at[cid, 0])


def _make_agg(compute_deg):
    mesh = plsc.VectorSubcoreMesh(core_axis_name="c", subcore_axis_name="s",
                                  num_cores=NC, num_subcores=NS)
    m_type = jax.ShapeDtypeStruct((NC, NP, D), jnp.float32)
    if compute_deg:
        out_type = [m_type, jax.ShapeDtypeStruct((NC, 1, NP), jnp.float32)]
    else:
        out_type = m_type
    scratch = [
        pltpu.VMEM_SHARED((NP, D), jnp.float32),
        pltpu.VMEM_SHARED((NP,), jnp.float32) if compute_deg else None,
        pltpu.VMEM((C, K), jnp.int32),
        pltpu.VMEM((C, K), jnp.int32),
        pltpu.VMEM((K, D), jnp.float32),
        pltpu.VMEM((K,), jnp.float32) if compute_deg else None,
        pltpu.SemaphoreType.DMA,
    ]
    scratch = [s for s in scratch if s is not None]

    if compute_deg:
        def body(x, src, dst, m_out, deg_out, acc, deg, sv, dv, rv, ov, sem):
            _agg_body(True, x, src, dst, m_out, deg_out, acc, deg, sv, dv, rv,
                      ov, sem)
    else:
        def body(x, src, dst, m_out, acc, sv, dv, rv, sem):
            _agg_body(False, x, src, dst, m_out, None, acc, None, sv, dv, rv,
                      None, sem)

    return pl.kernel(body, out_type=out_type, mesh=mesh, scratch_types=scratch,
                     name="gcn_agg_deg" if compute_deg else "gcn_agg")


_AGG_CACHE = {}


def _get_agg(compute_deg):
    if compute_deg not in _AGG_CACHE:
        _AGG_CACHE[compute_deg] = _make_agg(compute_deg)
    return _AGG_CACHE[compute_deg]


def _fused_layer_body(act, m_ref, deg_ref, w_ref, b_ref, out_ref):
    msum = m_ref[0] + m_ref[1]
    d = deg_ref[0] + deg_ref[1]
    inv = 1.0 / jnp.maximum(d, 1.0)
    y = jnp.dot(msum, w_ref[...], preferred_element_type=jnp.float32)
    y = y * inv + b_ref[...]
    if act:
        y = jnp.maximum(y, 0.0)
    out_ref[...] = y


def _make_fused_layer(act, rows_blk=1264):
    grid = (NP // rows_blk,)
    return pl.pallas_call(
        functools.partial(_fused_layer_body, act),
        grid=grid,
        in_specs=[
            pl.BlockSpec((NC, rows_blk, D), lambda i: (0, i, 0)),
            pl.BlockSpec((NC, rows_blk, 1), lambda i: (0, i, 0)),
            pl.BlockSpec((D, D), lambda i: (0, 0)),
            pl.BlockSpec((1, D), lambda i: (0, 0)),
        ],
        out_specs=pl.BlockSpec((rows_blk, D), lambda i: (i, 0)),
        out_shape=jax.ShapeDtypeStruct((NP, D), jnp.float32),
        name="gcn_fused_layer",
    )


_fused_relu = _make_fused_layer(True)
_fused_lin = _make_fused_layer(False)


def kernel(features, edge_index, W0, b0, W1, b1, W2, b2):
    src = edge_index[0].astype(jnp.int32).reshape(NW, C, K)
    dst = edge_index[1].astype(jnp.int32).reshape(NW, C, K)

    m0, deg = _get_agg(True)(features, src, dst)
    deg3 = deg[:, 0, :, None]

    W2p = jnp.zeros((D, D), jnp.float32).at[:, :40].set(W2)
    b2p = jnp.zeros((D,), jnp.float32).at[:40].set(b2)

    h1 = _fused_relu(m0, deg3, W0, b0[None, :])
    m1 = _get_agg(False)(h1, src, dst)
    h2 = _fused_relu(m1, deg3, W1, b1[None, :])
    m2 = _get_agg(False)(h2, src, dst)
    out = _fused_lin(m2, deg3, W2p, b2p[None, :])
    return out[:N_NODES, :40]
